# baseline (speedup 1.0000x reference)
import functools

import jax
import jax.numpy as jnp
from jax.experimental import pallas as pl
from jax.experimental.pallas import tpu as pltpu

_N_E = 8192
_E = 256
_B = 16
_T = 256
_LC = 0.2

_CB = 512
_NBLK = _N_E // _CB
_W = 512


def _ab_kernel(ks_ref, w_ref, wl_ref, ks3_ref, wf_ref,
               mi_ref, enc_ref, v_ref,
               s1_s, rv_ref, ri_ref, adv_s, p_s):
    i = pl.program_id(0)

    @pl.when(i == 0)
    def _():
        ks = ks_ref[...]
        s1_s[...] = jnp.sum(ks * ks, axis=1, keepdims=True)
        wl = wl_ref[...]
        s2l = jnp.sum(wl * wl, axis=1)
        mml = jax.lax.dot_general(ks, wl, (((1,), (1,)), ((), ())),
                                  preferred_element_type=jnp.float32)
        rv_ref[...] = s1_s[...] + s2l[None, :] - 2.0 * mml
        ri_ref[...] = jnp.full((ks.shape[0], 1), _N_E, jnp.int32)

    @pl.when(i < _NBLK)
    def _():
        ks = ks_ref[...]
        w = w_ref[...]
        s1 = s1_s[...]
        s2 = jnp.sum(w * w, axis=1)
        mm = jax.lax.dot_general(ks, w, (((1,), (1,)), ((), ())),
                                 preferred_element_type=jnp.float32)
        d = s1 + s2[None, :] - 2.0 * mm
        bval = jnp.min(d, axis=1, keepdims=True)
        iot = jax.lax.broadcasted_iota(jnp.int32, d.shape, 1)
        bidx = jnp.min(jnp.where(d == bval, iot, 2 ** 30), axis=1,
                       keepdims=True) + i * _CB
        rv = rv_ref[...]
        ri = ri_ref[...]
        upd = (bval < rv) | ((bval == rv) & (bidx < ri))
        rv_ref[...] = jnp.where(upd, bval, rv)
        ri_ref[...] = jnp.where(upd, bidx, ri)

    @pl.when(i == _NBLK)
    def _():
        mi_ref[...] = ri_ref[...]
        iot_j = jax.lax.broadcasted_iota(jnp.int32, (_B, _W), 1)
        iot_1w = jax.lax.broadcasted_iota(jnp.int32, (1, _W), 1)
        p0_rows = []
        j0_rows = []
        sb_rows = []
        for b in range(_B):
            e0 = jnp.clip(ri_ref[b * _T, 0], 0, _N_E - 1)
            s = jnp.minimum((e0 // 256) * 256, _N_E - _W)
            s = pl.multiple_of(s, 256)
            j0b = e0 - s
            wwin = wf_ref[pl.ds(s, _W), :]
            ks_b = ks3_ref[b]
            s1b = jnp.sum(ks_b * ks_b, axis=1, keepdims=True)
            s2b = jnp.sum(wwin * wwin, axis=1)
            mmb = jax.lax.dot_general(ks_b, wwin, (((1,), (1,)), ((), ())),
                                      preferred_element_type=jnp.float32)
            dw = s1b + s2b[None, :] - 2.0 * mmb
            dnext = jnp.concatenate([dw[:, 1:], dw[:, :1]], axis=1)
            adv = (dnext < dw) & (jax.lax.broadcasted_iota(
                jnp.int32, dw.shape, 1) < (_N_E - 1 - s))
            adv_s[:, b, :] = adv.astype(jnp.float32)
            p0_rows.append((iot_1w == j0b).astype(jnp.float32))
            j0_rows.append(jnp.zeros((1, 1), jnp.int32) + j0b)
            sb_rows.append(jnp.zeros((1, 1), jnp.int32) + s)

        p0 = jnp.concatenate(p0_rows, axis=0)
        j0 = jnp.concatenate(j0_rows, axis=0)
        sbcol = jnp.concatenate(sb_rows, axis=0)
        p_s[0] = p0

        def body(t, carry):
            p, j = carry
            advrow = adv_s[pl.ds(t, 1)].reshape(_B, _W)
            a = jnp.sum(p * advrow, axis=1, keepdims=True)
            pshift = jnp.concatenate([p[:, :1] * 0.0, p[:, :-1]], axis=1)
            p = jnp.where(a > 0.0, pshift, p)
            j = j + a.astype(jnp.int32)
            p_s[pl.ds(t, 1)] = p.reshape(1, _B, _W)
            return (p, j)

        pT, jT = jax.lax.fori_loop(1, _T, body, (p0, j0))

        pall = p_s[...]
        jf = jnp.sum(pall * iot_j[None].astype(jnp.float32), axis=2)
        enc_ref[...] = jnp.swapaxes(jf.astype(jnp.int32), 0, 1) + sbcol
        v_ref[...] = jnp.max(jT - j0)[None, None]


def _ab_pass(ksf, key_soft, W):
    bt = ksf.shape[0]
    return pl.pallas_call(
        _ab_kernel,
        grid=(_NBLK + 1,),
        in_specs=[
            pl.BlockSpec((bt, _E), lambda i: (0, 0)),
            pl.BlockSpec((_CB, _E), lambda i: (jnp.minimum(i, _NBLK - 1), 0)),
            pl.BlockSpec((1, _E), lambda i: (0, 0)),
            pl.BlockSpec((_B, _T, _E), lambda i: (0, 0, 0)),
            pl.BlockSpec((_N_E + 1, _E), lambda i: (0, 0)),
        ],
        out_specs=[
            pl.BlockSpec((bt, 1), lambda i: (0, 0)),
            pl.BlockSpec((_B, _T), lambda i: (0, 0)),
            pl.BlockSpec((1, 1), lambda i: (0, 0)),
        ],
        out_shape=[
            jax.ShapeDtypeStruct((bt, 1), jnp.int32),
            jax.ShapeDtypeStruct((_B, _T), jnp.int32),
            jax.ShapeDtypeStruct((1, 1), jnp.int32),
        ],
        scratch_shapes=[
            pltpu.VMEM((bt, 1), jnp.float32),
            pltpu.VMEM((bt, 1), jnp.float32),
            pltpu.VMEM((bt, 1), jnp.int32),
            pltpu.VMEM((_T, _B, _W), jnp.float32),
            pltpu.VMEM((_T, _B, _W), jnp.float32),
        ],
    )(ksf, W, W[_N_E:], key_soft, W)


def _gather3(W, idx_h, idx_n, idx_m):
    from jax.experimental.pallas import tpu_sc as plsc
    from jax import lax
    info = plsc.get_sparse_core_info()
    nc, ns = info.num_cores, info.num_subcores
    nw = nc * ns
    bt = idx_h.shape[0]
    bpw = bt // nw
    mesh = plsc.VectorSubcoreMesh(core_axis_name="c", subcore_axis_name="s")

    @functools.partial(
        pl.kernel, mesh=mesh,
        out_type=[jax.ShapeDtypeStruct((bt, _E), jnp.float32)] * 3,
        scratch_types=[
            pltpu.VMEM((bpw,), jnp.int32),
            pltpu.VMEM((bpw,), jnp.int32),
            pltpu.VMEM((bpw,), jnp.int32),
            pltpu.VMEM((bpw, _E), jnp.float32),
            pltpu.VMEM((bpw, _E), jnp.float32),
            pltpu.VMEM((bpw, _E), jnp.float32),
            pltpu.SemaphoreType.DMA,
        ],
    )
    def k(w_hbm, ih_hbm, in_hbm, im_hbm, oh_hbm, on_hbm, om_hbm,
          i0, i1, i2, r0, r1, r2, sem):
        wid = lax.axis_index("s") * nc + lax.axis_index("c")
        base = wid * bpw
        idxs = (i0, i1, i2)
        rows = (r0, r1, r2)
        for src, iv in zip((ih_hbm, in_hbm, im_hbm), idxs):
            pltpu.sync_copy(src.at[pl.ds(base, bpw)], iv)
        copies = [pltpu.async_copy(w_hbm.at[iv], rv, sem)
                  for iv, rv in zip(idxs, rows)]
        for c in copies:
            c.wait()
        for rv, dst in zip(rows, (oh_hbm, on_hbm, om_hbm)):
            pltpu.sync_copy(rv, dst.at[pl.ds(base, bpw)])

    return k(W, idx_h, idx_n, idx_m)


def _rowsum(x):
    u = x[:, :128] + x[:, 128:]
    acc = u[:, 0:8]
    for t in range(1, 16):
        acc = acc + u[:, 8 * t: 8 * t + 8]
    q = acc[:, 0:4] + acc[:, 4:8]
    r = q[:, 0:2] + q[:, 2:4]
    return r[:, 0:1] + r[:, 1:2]


def _loss_kernel(ks_ref, kh_ref, kn_ref, km_ref, kout_ref, lh_ref, ln_ref):
    ks = ks_ref[...]
    kh = kh_ref[...]
    kout_ref[...] = ks + (kh - ks)
    rh = _rowsum((ks - kh) ** 2)
    rn = _rowsum((ks - kn_ref[...]) ** 2)
    rm = _rowsum((ks - km_ref[...]) ** 2)
    base_h = rh * _LC + rh
    base_n = rn * _LC + rn
    lmi = rm + rm * _LC
    lh_ref[...] = base_h - jnp.where(lmi < base_h, lmi, 0.0)
    ln_ref[...] = base_n - jnp.where(lmi < base_n, lmi, 0.0)


def _loss_pass(ksf, kh, kn, km):
    bt = ksf.shape[0]
    blk = 2048
    return pl.pallas_call(
        _loss_kernel,
        grid=(bt // blk,),
        in_specs=[pl.BlockSpec((blk, _E), lambda i: (i, 0))] * 4,
        out_specs=[pl.BlockSpec((blk, _E), lambda i: (i, 0)),
                   pl.BlockSpec((blk, 1), lambda i: (i, 0)),
                   pl.BlockSpec((blk, 1), lambda i: (i, 0))],
        out_shape=[jax.ShapeDtypeStruct((bt, _E), jnp.float32)]
        + [jax.ShapeDtypeStruct((bt, 1), jnp.float32)] * 2,
    )(ksf, kh, kn, km)


def kernel(key_soft, W):
    Bx, Tx, e_dim = key_soft.shape
    n_e = W.shape[0] - 1

    ksf = key_soft.reshape(-1, e_dim)
    min_idx, encoding_indices, v11 = _ab_pass(ksf, key_soft, W)
    min_indices = min_idx[:, 0]
    v = v11.reshape(())

    eif = encoding_indices.reshape(-1)
    kh, kn, km = _gather3(W, eif, jnp.clip(eif + 1, 0, n_e - 1), min_indices)
    khard, lh, ln = _loss_pass(ksf, kh, kn, km)

    key_hard = khard.reshape(key_soft.shape)
    loss_here = lh[:, 0].reshape(Bx, Tx)
    loss_next = ln[:, 0].reshape(Bx, Tx)
    return (key_hard, encoding_indices, v, loss_here, loss_next)


# speedup vs baseline: 1.0118x; 1.0118x over previous
import functools

import jax
import jax.numpy as jnp
from jax.experimental import pallas as pl
from jax.experimental.pallas import tpu as pltpu

_N_E = 8192
_E = 256
_B = 16
_T = 256
_LC = 0.2

_CB = 512
_NBLK = _N_E // _CB
_W = 512


def _ab_kernel(ks_ref, w_ref, wl_ref, ks3_ref, wf_ref,
               mi_ref, enc_ref, v_ref,
               s1_s, rv_ref, ri_ref, adv_s, p_s):
    i = pl.program_id(0)

    @pl.when(i == 0)
    def _():
        ks = ks_ref[...]
        s1_s[...] = jnp.sum(ks * ks, axis=1, keepdims=True)
        wl = wl_ref[...]
        s2l = jnp.sum(wl * wl, axis=1)
        mml = jax.lax.dot_general(ks, wl, (((1,), (1,)), ((), ())),
                                  preferred_element_type=jnp.float32)
        rv_ref[...] = s1_s[...] + s2l[None, :] - 2.0 * mml
        ri_ref[...] = jnp.full((ks.shape[0], 1), _N_E, jnp.int32)

    @pl.when(i < _NBLK)
    def _():
        ks = ks_ref[...]
        w = w_ref[...]
        s1 = s1_s[...]
        s2 = jnp.sum(w * w, axis=1)
        mm = jax.lax.dot_general(ks, w, (((1,), (1,)), ((), ())),
                                 preferred_element_type=jnp.float32)
        d = s1 + s2[None, :] - 2.0 * mm
        bval = jnp.min(d, axis=1, keepdims=True)
        iot = jax.lax.broadcasted_iota(jnp.int32, d.shape, 1)
        bidx = jnp.min(jnp.where(d == bval, iot, 2 ** 30), axis=1,
                       keepdims=True) + i * _CB
        rv = rv_ref[...]
        ri = ri_ref[...]
        upd = (bval < rv) | ((bval == rv) & (bidx < ri))
        rv_ref[...] = jnp.where(upd, bval, rv)
        ri_ref[...] = jnp.where(upd, bidx, ri)

    @pl.when(i == _NBLK)
    def _():
        mi_ref[...] = ri_ref[...]
        iot_j = jax.lax.broadcasted_iota(jnp.int32, (_B, _W), 1)
        iot_1w = jax.lax.broadcasted_iota(jnp.int32, (1, _W), 1)
        p0_rows = []
        j0_rows = []
        sb_rows = []
        for b in range(_B):
            e0 = jnp.clip(ri_ref[b * _T, 0], 0, _N_E - 1)
            s = jnp.minimum((e0 // 256) * 256, _N_E - _W)
            s = pl.multiple_of(s, 256)
            j0b = e0 - s
            wwin = wf_ref[pl.ds(s, _W), :]
            ks_b = ks3_ref[b]
            s1b = jnp.sum(ks_b * ks_b, axis=1, keepdims=True)
            s2b = jnp.sum(wwin * wwin, axis=1)
            mmb = jax.lax.dot_general(ks_b, wwin, (((1,), (1,)), ((), ())),
                                      preferred_element_type=jnp.float32)
            dw = s1b + s2b[None, :] - 2.0 * mmb
            dnext = jnp.concatenate([dw[:, 1:], dw[:, :1]], axis=1)
            adv = (dnext < dw) & (jax.lax.broadcasted_iota(
                jnp.int32, dw.shape, 1) < (_N_E - 1 - s))
            adv_s[:, b, :] = adv.astype(jnp.float32)
            p0_rows.append((iot_1w == j0b).astype(jnp.float32))
            j0_rows.append(jnp.zeros((1, 1), jnp.int32) + j0b)
            sb_rows.append(jnp.zeros((1, 1), jnp.int32) + s)

        p0 = jnp.concatenate(p0_rows, axis=0)
        j0 = jnp.concatenate(j0_rows, axis=0)
        sbcol = jnp.concatenate(sb_rows, axis=0)
        p_s[0] = p0

        def body(t, carry):
            p, j = carry
            advrow = adv_s[pl.ds(t, 1)].reshape(_B, _W)
            a = jnp.sum(p * advrow, axis=1, keepdims=True)
            pshift = jnp.concatenate([p[:, :1] * 0.0, p[:, :-1]], axis=1)
            p = jnp.where(a > 0.0, pshift, p)
            j = j + a.astype(jnp.int32)
            p_s[pl.ds(t, 1)] = p.reshape(1, _B, _W)
            return (p, j)

        pT, jT = jax.lax.fori_loop(1, _T, body, (p0, j0))

        pall = p_s[...]
        jf = jnp.sum(pall * iot_j[None].astype(jnp.float32), axis=2)
        enc_ref[...] = jnp.swapaxes(jf.astype(jnp.int32), 0, 1) + sbcol
        v_ref[...] = jnp.max(jT - j0)[None, None]


def _ab_pass(ksf, key_soft, W):
    bt = ksf.shape[0]
    return pl.pallas_call(
        _ab_kernel,
        grid=(_NBLK + 1,),
        in_specs=[
            pl.BlockSpec((bt, _E), lambda i: (0, 0)),
            pl.BlockSpec((_CB, _E), lambda i: (jnp.minimum(i, _NBLK - 1), 0)),
            pl.BlockSpec((1, _E), lambda i: (0, 0)),
            pl.BlockSpec((_B, _T, _E), lambda i: (0, 0, 0)),
            pl.BlockSpec((_N_E + 1, _E), lambda i: (0, 0)),
        ],
        out_specs=[
            pl.BlockSpec((bt, 1), lambda i: (0, 0)),
            pl.BlockSpec((_B, _T), lambda i: (0, 0)),
            pl.BlockSpec((1, 1), lambda i: (0, 0)),
        ],
        out_shape=[
            jax.ShapeDtypeStruct((bt, 1), jnp.int32),
            jax.ShapeDtypeStruct((_B, _T), jnp.int32),
            jax.ShapeDtypeStruct((1, 1), jnp.int32),
        ],
        scratch_shapes=[
            pltpu.VMEM((bt, 1), jnp.float32),
            pltpu.VMEM((bt, 1), jnp.float32),
            pltpu.VMEM((bt, 1), jnp.int32),
            pltpu.VMEM((_T, _B, _W), jnp.float32),
            pltpu.VMEM((_T, _B, _W), jnp.float32),
        ],
    )(ksf, W, W[_N_E:], key_soft, W)


def _gather3(W, idx_h, idx_n, idx_m):
    from jax.experimental.pallas import tpu_sc as plsc
    from jax import lax
    info = plsc.get_sparse_core_info()
    nc, ns = info.num_cores, info.num_subcores
    nw = nc * ns
    bt = idx_h.shape[0]
    bpw = bt // nw
    mesh = plsc.VectorSubcoreMesh(core_axis_name="c", subcore_axis_name="s")

    @functools.partial(
        pl.kernel, mesh=mesh,
        out_type=[jax.ShapeDtypeStruct((bt, _E), jnp.float32)] * 3,
        scratch_types=[
            pltpu.VMEM((bpw,), jnp.int32),
            pltpu.VMEM((bpw,), jnp.int32),
            pltpu.VMEM((bpw,), jnp.int32),
            pltpu.VMEM((bpw, _E), jnp.float32),
            pltpu.VMEM((bpw, _E), jnp.float32),
            pltpu.VMEM((bpw, _E), jnp.float32),
            pltpu.SemaphoreType.DMA,
        ],
    )
    def k(w_hbm, ih_hbm, in_hbm, im_hbm, oh_hbm, on_hbm, om_hbm,
          i0, i1, i2, r0, r1, r2, sem):
        wid = lax.axis_index("s") * nc + lax.axis_index("c")
        base = wid * bpw
        idxs = (i0, i1, i2)
        rows = (r0, r1, r2)
        for src, iv in zip((ih_hbm, in_hbm, im_hbm), idxs):
            pltpu.sync_copy(src.at[pl.ds(base, bpw)], iv)
        copies = [pltpu.async_copy(w_hbm.at[iv], rv, sem)
                  for iv, rv in zip(idxs, rows)]
        for c in copies:
            c.wait()
        for rv, dst in zip(rows, (oh_hbm, on_hbm, om_hbm)):
            pltpu.sync_copy(rv, dst.at[pl.ds(base, bpw)])

    return k(W, idx_h, idx_n, idx_m)


def _rowsum(x):
    u = x[:, :128] + x[:, 128:]
    acc = u[:, 0:8]
    for t in range(1, 16):
        acc = acc + u[:, 8 * t: 8 * t + 8]
    q = acc[:, 0:4] + acc[:, 4:8]
    r = q[:, 0:2] + q[:, 2:4]
    return r[:, 0:1] + r[:, 1:2]


def _loss_kernel(ks_ref, kh_ref, kn_ref, km_ref, kout_ref, lh_ref, ln_ref):
    ks = ks_ref[...]
    kh = kh_ref[...]
    kout_ref[...] = ks + (kh - ks)
    rh = _rowsum((ks - kh) ** 2)
    rn = _rowsum((ks - kn_ref[...]) ** 2)
    rm = _rowsum((ks - km_ref[...]) ** 2)
    base_h = rh * _LC + rh
    base_n = rn * _LC + rn
    lmi = rm + rm * _LC
    lh_ref[...] = base_h - jnp.where(lmi < base_h, lmi, 0.0)
    ln_ref[...] = base_n - jnp.where(lmi < base_n, lmi, 0.0)


def _loss_pass(ksf, kh, kn, km):
    bt = ksf.shape[0]
    blk = 1024
    return pl.pallas_call(
        _loss_kernel,
        grid=(bt // blk,),
        in_specs=[pl.BlockSpec((blk, _E), lambda i: (i, 0))] * 4,
        out_specs=[pl.BlockSpec((blk, _E), lambda i: (i, 0)),
                   pl.BlockSpec((blk, 1), lambda i: (i, 0)),
                   pl.BlockSpec((blk, 1), lambda i: (i, 0))],
        out_shape=[jax.ShapeDtypeStruct((bt, _E), jnp.float32)]
        + [jax.ShapeDtypeStruct((bt, 1), jnp.float32)] * 2,
    )(ksf, kh, kn, km)


def kernel(key_soft, W):
    Bx, Tx, e_dim = key_soft.shape
    n_e = W.shape[0] - 1

    ksf = key_soft.reshape(-1, e_dim)
    min_idx, encoding_indices, v11 = _ab_pass(ksf, key_soft, W)
    min_indices = min_idx[:, 0]
    v = v11.reshape(())

    eif = encoding_indices.reshape(-1)
    kh, kn, km = _gather3(W, eif, jnp.clip(eif + 1, 0, n_e - 1), min_indices)
    khard, lh, ln = _loss_pass(ksf, kh, kn, km)

    key_hard = khard.reshape(key_soft.shape)
    loss_here = lh[:, 0].reshape(Bx, Tx)
    loss_next = ln[:, 0].reshape(Bx, Tx)
    return (key_hard, encoding_indices, v, loss_here, loss_next)
